# baseline (speedup 1.0000x reference)
import functools

import jax
import jax.numpy as jnp
from jax import lax
from jax.experimental import pallas as pl
from jax.experimental.pallas import tpu as pltpu
from jax.experimental.pallas import tpu_sc as plsc

N_NODES = 10000
N_EDGES = 320000
D = 128

NC = 2
NS = 16
NW = NC * NS

EB = 128
EROWS = 2560
E_PAD = EROWS * EB
ROWS_PER_W = EROWS // NW

N_PAD = 10240
NCH = N_PAD // NS


def _deg_body(dst_hbm, ones_hbm, zeros_hbm, degp_hbm, dstv, onesv, deg_sh):
    c = lax.axis_index("c")
    s = lax.axis_index("s")
    gid = c * NS + s
    row0 = gid * ROWS_PER_W
    pltpu.sync_copy(dst_hbm.at[pl.ds(row0, ROWS_PER_W)], dstv)
    pltpu.sync_copy(ones_hbm, onesv)
    n0 = s * NCH
    pltpu.sync_copy(zeros_hbm.at[pl.ds(n0, NCH)], deg_sh.at[pl.ds(n0, NCH)])
    plsc.subcore_barrier()

    def step(i, carry):
        pltpu.sync_copy(onesv, deg_sh.at[dstv.at[i]], add=True)
        return carry

    lax.fori_loop(0, ROWS_PER_W, step, 0)
    plsc.subcore_barrier()
    pltpu.sync_copy(deg_sh.at[pl.ds(n0, NCH)],
                    degp_hbm.at[c, pl.ds(n0, NCH)])


@functools.cache
def _deg_kernel():
    return pl.kernel(
        _deg_body,
        out_type=jax.ShapeDtypeStruct((NC, N_PAD, D), jnp.float32),
        mesh=plsc.VectorSubcoreMesh(core_axis_name="c", subcore_axis_name="s",
                                    num_cores=NC, num_subcores=NS),
        scratch_types=[
            pltpu.VMEM((ROWS_PER_W, EB), jnp.int32),
            pltpu.VMEM((EB, D), jnp.float32),
            pltpu.VMEM_SHARED((N_PAD, D), jnp.float32),
        ],
    )


NBUF = 2
IB = 40


ROWS_C0 = 1280
ROWS_C1 = EROWS - ROWS_C0


def _edge_loop(row0, nblocks, src_hbm, dst_hbm, y_hbm,
               srcv, dstv, rows, gsems, ssems, agg_sh):
    for k in range(nblocks):
        pltpu.sync_copy(src_hbm.at[pl.ds(row0 + k * IB, IB)], srcv)
        pltpu.sync_copy(dst_hbm.at[pl.ds(row0 + k * IB, IB)], dstv)

        def gath(j, b):
            for h in range(0, EB, EB // 2):
                pltpu.async_copy(y_hbm.at[srcv.at[j, pl.ds(h, EB // 2)]],
                                 rows.at[b, pl.ds(h, EB // 2)], gsems[b])

        for b in range(NBUF):
            gath(b, b)

        def lap(i, carry):
            base = i * NBUF
            for b in range(NBUF):
                j = base + b
                pltpu.make_async_copy(y_hbm.at[srcv.at[j]], rows.at[b],
                                      gsems[b]).wait()
                pltpu.async_copy(rows.at[b], agg_sh.at[dstv.at[j]], ssems[b],
                                 add=True)
                nxt = j + NBUF

                @pl.when(nxt < IB)
                def _():
                    pltpu.make_async_copy(rows.at[b], agg_sh.at[dstv.at[j]],
                                          ssems[b]).wait()
                    gath(nxt, b)
            return carry

        lax.fori_loop(0, IB // NBUF, lap, 0)
        for b in range(NBUF):
            pltpu.make_async_copy(rows.at[b], agg_sh.at[dstv.at[0]],
                                  ssems[b]).wait()


def _agg_body(src_hbm, dst_hbm, y_hbm, zeros_hbm, aggp_hbm,
              srcv, dstv, rows, gsem0, gsem1, ssem0, ssem1, agg_sh):
    gsems = (gsem0, gsem1)
    ssems = (ssem0, ssem1)
    c = lax.axis_index("c")
    s = lax.axis_index("s")
    n0 = s * NCH
    pltpu.sync_copy(zeros_hbm.at[pl.ds(n0, NCH)], agg_sh.at[pl.ds(n0, NCH)])
    plsc.subcore_barrier()

    if ROWS_C0 > 0:
        @pl.when(c == 0)
        def _c0():
            _edge_loop(s * (ROWS_C0 // NS), ROWS_C0 // NS // IB,
                       src_hbm, dst_hbm, y_hbm, srcv, dstv, rows,
                       gsems, ssems, agg_sh)

    if ROWS_C1 > 0:
        @pl.when(c == 1)
        def _c1():
            _edge_loop(ROWS_C0 + s * (ROWS_C1 // NS), ROWS_C1 // NS // IB,
                       src_hbm, dst_hbm, y_hbm, srcv, dstv, rows,
                       gsems, ssems, agg_sh)

    plsc.subcore_barrier()
    pltpu.sync_copy(agg_sh.at[pl.ds(n0, NCH)],
                    aggp_hbm.at[c, pl.ds(n0, NCH)])


@functools.cache
def _agg_kernel():
    return pl.kernel(
        _agg_body,
        out_type=jax.ShapeDtypeStruct((NC, N_PAD, D), jnp.float32),
        mesh=plsc.VectorSubcoreMesh(core_axis_name="c", subcore_axis_name="s",
                                    num_cores=NC, num_subcores=NS),
        scratch_types=[
            pltpu.VMEM((IB, EB), jnp.int32),
            pltpu.VMEM((IB, EB), jnp.int32),
            pltpu.VMEM((NBUF, EB, D), jnp.float32),
            pltpu.SemaphoreType.DMA,
            pltpu.SemaphoreType.DMA,
            pltpu.SemaphoreType.DMA,
            pltpu.SemaphoreType.DMA,
            pltpu.VMEM_SHARED((N_PAD, D), jnp.float32),
        ],
    )


BM = 1024
BF = 1000


def _xw_body(x_ref, w_ref, xw_ref):
    xw_ref[...] = jnp.dot(x_ref[...], w_ref[...],
                          preferred_element_type=jnp.float32)


def _xw(xp, W):
    return pl.pallas_call(
        _xw_body,
        grid=(N_PAD // BM,),
        in_specs=[
            pl.BlockSpec((BM, D), lambda i: (i, 0)),
            pl.BlockSpec((D, D), lambda i: (0, 0)),
        ],
        out_specs=pl.BlockSpec((BM, D), lambda i: (i, 0)),
        out_shape=jax.ShapeDtypeStruct((N_PAD, D), jnp.float32),
    )(xp, W)


def _scale_body(degp_ref, xw_ref, y_ref):
    deg = degp_ref[0, :, 0:1] + degp_ref[1, :, 0:1] + 1.0
    dis = lax.rsqrt(deg)
    y_ref[...] = xw_ref[...] * dis


def _scale(degp, xw):
    return pl.pallas_call(
        _scale_body,
        grid=(N_PAD // BM,),
        in_specs=[
            pl.BlockSpec((NC, BM, D), lambda i: (0, i, 0)),
            pl.BlockSpec((BM, D), lambda i: (i, 0)),
        ],
        out_specs=pl.BlockSpec((BM, D), lambda i: (i, 0)),
        out_shape=jax.ShapeDtypeStruct((N_PAD, D), jnp.float32),
    )(degp, xw)


def _final_body(degp_ref, aggp_ref, y_ref, b_ref, out_ref):
    deg = degp_ref[0, :, 0:1] + degp_ref[1, :, 0:1] + 1.0
    dis = lax.rsqrt(deg)
    acc = aggp_ref[0] + aggp_ref[1] + y_ref[...]
    out_ref[...] = acc * dis + b_ref[...]


def _final(degp, aggp, y, b2):
    return pl.pallas_call(
        _final_body,
        grid=(N_NODES // BF,),
        in_specs=[
            pl.BlockSpec((NC, BF, D), lambda i: (0, i, 0)),
            pl.BlockSpec((NC, BF, D), lambda i: (0, i, 0)),
            pl.BlockSpec((BF, D), lambda i: (i, 0)),
            pl.BlockSpec((1, D), lambda i: (0, 0)),
        ],
        out_specs=pl.BlockSpec((BF, D), lambda i: (i, 0)),
        out_shape=jax.ShapeDtypeStruct((N_NODES, D), jnp.float32),
    )(degp, aggp, y, b2)


def kernel(x, edge_index, W, b):
    ei = edge_index.astype(jnp.int32)
    pad = N_NODES + jnp.arange(E_PAD - N_EDGES, dtype=jnp.int32) % (
        N_PAD - N_NODES)
    src = jnp.concatenate([ei[0], pad]).reshape(EROWS, EB)
    dst = jnp.concatenate([ei[1], pad]).reshape(EROWS, EB)
    xp = jnp.pad(x, ((0, N_PAD - N_NODES), (0, 0)))
    onesD = jnp.ones((EB, D), jnp.float32)
    zerosD = jnp.zeros((N_PAD, D), jnp.float32)

    degp = _deg_kernel()(dst, onesD, zerosD)
    xw = _xw(xp, W)
    y = _scale(degp, xw)
    aggp = _agg_kernel()(src, dst, y, zerosD)
    out = _final(degp, aggp, y, b.reshape(1, D))
    return out


# speedup vs baseline: 1.0196x; 1.0196x over previous
import functools

import jax
import jax.numpy as jnp
import numpy as np
from jax import lax
from jax.experimental import pallas as pl
from jax.experimental.pallas import tpu as pltpu
from jax.experimental.pallas import tpu_sc as plsc

N_NODES = 10000
N_EDGES = 320000
D = 128

NC = 2
NS = 16
NW = NC * NS

EB = 128
EROWS = 2560
E_PAD = EROWS * EB
ROWS_PER_W = EROWS // NW

N_PAD = 10240
NCH = N_PAD // NS


def _zero_slice(zbuf, sh, n0):
    def zrow(i, carry):
        r = i // 8
        k = i - r * 8
        zbuf[r, pl.ds(k * 16, 16)] = jnp.zeros((16,), jnp.float32)
        return carry

    lax.fori_loop(0, 128 * 8, zrow, 0)
    for r in range(NCH // 128):
        pltpu.sync_copy(zbuf, sh.at[pl.ds(n0 + r * 128, 128)])


def _deg_body(dst_hbm, ones_hbm, degp_hbm, dstv, onesv, zbuf, deg_sh):
    c = lax.axis_index("c")
    s = lax.axis_index("s")
    gid = c * NS + s
    row0 = gid * ROWS_PER_W
    pltpu.sync_copy(dst_hbm.at[pl.ds(row0, ROWS_PER_W)], dstv)
    pltpu.sync_copy(ones_hbm, onesv)
    n0 = s * NCH
    _zero_slice(zbuf, deg_sh, n0)
    plsc.subcore_barrier()

    def step(i, carry):
        pltpu.sync_copy(onesv, deg_sh.at[dstv.at[i]], add=True)
        return carry

    lax.fori_loop(0, ROWS_PER_W, step, 0)
    plsc.subcore_barrier()
    pltpu.sync_copy(deg_sh.at[pl.ds(n0, NCH)],
                    degp_hbm.at[c, pl.ds(n0, NCH)])


@functools.cache
def _deg_kernel():
    return pl.kernel(
        _deg_body,
        out_type=jax.ShapeDtypeStruct((NC, N_PAD, D), jnp.float32),
        mesh=plsc.VectorSubcoreMesh(core_axis_name="c", subcore_axis_name="s",
                                    num_cores=NC, num_subcores=NS),
        scratch_types=[
            pltpu.VMEM((ROWS_PER_W, EB), jnp.int32),
            pltpu.VMEM((EB, D), jnp.float32),
            pltpu.VMEM((128, D), jnp.float32),
            pltpu.VMEM_SHARED((N_PAD, D), jnp.float32),
        ],
    )


NBUF = 2
IB = 40


ROWS_C0 = 1280
ROWS_C1 = EROWS - ROWS_C0


def _edge_loop(row0, nblocks, src_hbm, dst_hbm, y_hbm,
               srcv, dstv, rows, gsems, ssems, agg_sh):
    for k in range(nblocks):
        pltpu.sync_copy(src_hbm.at[pl.ds(row0 + k * IB, IB)], srcv)
        pltpu.sync_copy(dst_hbm.at[pl.ds(row0 + k * IB, IB)], dstv)

        def gath(j, b):
            for h in range(0, EB, EB // 2):
                pltpu.async_copy(y_hbm.at[srcv.at[j, pl.ds(h, EB // 2)]],
                                 rows.at[b, pl.ds(h, EB // 2)], gsems[b])

        for b in range(NBUF):
            gath(b, b)

        def lap(i, carry):
            base = i * NBUF
            for b in range(NBUF):
                j = base + b
                pltpu.make_async_copy(y_hbm.at[srcv.at[j]], rows.at[b],
                                      gsems[b]).wait()
                pltpu.async_copy(rows.at[b], agg_sh.at[dstv.at[j]], ssems[b],
                                 add=True)
                nxt = j + NBUF

                @pl.when(nxt < IB)
                def _():
                    pltpu.make_async_copy(rows.at[b], agg_sh.at[dstv.at[j]],
                                          ssems[b]).wait()
                    gath(nxt, b)
            return carry

        lax.fori_loop(0, IB // NBUF, lap, 0)
        for b in range(NBUF):
            pltpu.make_async_copy(rows.at[b], agg_sh.at[dstv.at[0]],
                                  ssems[b]).wait()


def _agg_body(src_hbm, dst_hbm, y_hbm, aggp_hbm,
              srcv, dstv, rows, gsem0, gsem1, ssem0, ssem1, agg_sh):
    gsems = (gsem0, gsem1)
    ssems = (ssem0, ssem1)
    c = lax.axis_index("c")
    s = lax.axis_index("s")
    n0 = s * NCH
    _zero_slice(rows.at[0], agg_sh, n0)
    plsc.subcore_barrier()

    if ROWS_C0 > 0:
        @pl.when(c == 0)
        def _c0():
            _edge_loop(s * (ROWS_C0 // NS), ROWS_C0 // NS // IB,
                       src_hbm, dst_hbm, y_hbm, srcv, dstv, rows,
                       gsems, ssems, agg_sh)

    if ROWS_C1 > 0:
        @pl.when(c == 1)
        def _c1():
            _edge_loop(ROWS_C0 + s * (ROWS_C1 // NS), ROWS_C1 // NS // IB,
                       src_hbm, dst_hbm, y_hbm, srcv, dstv, rows,
                       gsems, ssems, agg_sh)

    plsc.subcore_barrier()
    pltpu.sync_copy(agg_sh.at[pl.ds(n0, NCH)],
                    aggp_hbm.at[c, pl.ds(n0, NCH)])


@functools.cache
def _agg_kernel():
    return pl.kernel(
        _agg_body,
        out_type=jax.ShapeDtypeStruct((NC, N_PAD, D), jnp.float32),
        mesh=plsc.VectorSubcoreMesh(core_axis_name="c", subcore_axis_name="s",
                                    num_cores=NC, num_subcores=NS),
        scratch_types=[
            pltpu.VMEM((IB, EB), jnp.int32),
            pltpu.VMEM((IB, EB), jnp.int32),
            pltpu.VMEM((NBUF, EB, D), jnp.float32),
            pltpu.SemaphoreType.DMA,
            pltpu.SemaphoreType.DMA,
            pltpu.SemaphoreType.DMA,
            pltpu.SemaphoreType.DMA,
            pltpu.VMEM_SHARED((N_PAD, D), jnp.float32),
        ],
    )


BM = 1024
BF = 1000


def _xw_body(x_ref, w_ref, xw_ref):
    xw_ref[...] = jnp.dot(x_ref[...], w_ref[...],
                          preferred_element_type=jnp.float32)


def _xw(x, W):
    return pl.pallas_call(
        _xw_body,
        grid=(N_NODES // BF,),
        in_specs=[
            pl.BlockSpec((BF, D), lambda i: (i, 0)),
            pl.BlockSpec((D, D), lambda i: (0, 0)),
        ],
        out_specs=pl.BlockSpec((BF, D), lambda i: (i, 0)),
        out_shape=jax.ShapeDtypeStruct((N_PAD, D), jnp.float32),
    )(x, W)


def _scale_body(degp_ref, xw_ref, y_ref):
    deg = degp_ref[0, :, 0:1] + degp_ref[1, :, 0:1] + 1.0
    dis = lax.rsqrt(deg)
    y_ref[...] = xw_ref[...] * dis


def _scale(degp, xw):
    return pl.pallas_call(
        _scale_body,
        grid=(N_NODES // BF,),
        in_specs=[
            pl.BlockSpec((NC, BF, D), lambda i: (0, i, 0)),
            pl.BlockSpec((BF, D), lambda i: (i, 0)),
        ],
        out_specs=pl.BlockSpec((BF, D), lambda i: (i, 0)),
        out_shape=jax.ShapeDtypeStruct((N_PAD, D), jnp.float32),
    )(degp, xw)


def _final_body(degp_ref, aggp_ref, y_ref, b_ref, out_ref):
    deg = degp_ref[0, :, 0:1] + degp_ref[1, :, 0:1] + 1.0
    dis = lax.rsqrt(deg)
    acc = aggp_ref[0] + aggp_ref[1] + y_ref[...]
    out_ref[...] = acc * dis + b_ref[...]


def _final(degp, aggp, y, b2):
    return pl.pallas_call(
        _final_body,
        grid=(N_NODES // BF,),
        in_specs=[
            pl.BlockSpec((NC, BF, D), lambda i: (0, i, 0)),
            pl.BlockSpec((NC, BF, D), lambda i: (0, i, 0)),
            pl.BlockSpec((BF, D), lambda i: (i, 0)),
            pl.BlockSpec((1, D), lambda i: (0, 0)),
        ],
        out_specs=pl.BlockSpec((BF, D), lambda i: (i, 0)),
        out_shape=jax.ShapeDtypeStruct((N_NODES, D), jnp.float32),
    )(degp, aggp, y, b2)


_PAD_IDX = np.asarray(
    N_NODES + np.arange(E_PAD - N_EDGES, dtype=np.int32) % (N_PAD - N_NODES),
    dtype=np.int32)
_ONES_D = np.ones((EB, D), np.float32)


def kernel(x, edge_index, W, b):
    ei = edge_index.astype(jnp.int32)
    pad = jnp.asarray(_PAD_IDX)
    src = jnp.concatenate([ei[0], pad]).reshape(EROWS, EB)
    dst = jnp.concatenate([ei[1], pad]).reshape(EROWS, EB)
    onesD = jnp.asarray(_ONES_D)

    degp = _deg_kernel()(dst, onesD)
    xw = _xw(x, W)
    y = _scale(degp, xw)
    aggp = _agg_kernel()(src, dst, y)
    out = _final(degp, aggp, y, b.reshape(1, D))
    return out


# speedup vs baseline: 1.2922x; 1.2674x over previous
import functools

import jax
import jax.numpy as jnp
import numpy as np
from jax import lax
from jax.experimental import pallas as pl
from jax.experimental.pallas import tpu as pltpu
from jax.experimental.pallas import tpu_sc as plsc

N_NODES = 10000
N_EDGES = 320000
D = 128

NC = 2
NS = 16
NW = NC * NS

EB = 128
EROWS = 2560
E_PAD = EROWS * EB
ROWS_PER_W = EROWS // NW

N_PAD = 10240
NCH = N_PAD // NS


def _zero_slice(zbuf, sh, n0):
    def zrow(i, carry):
        r = i // 8
        k = i - r * 8
        zbuf[r, pl.ds(k * 16, 16)] = jnp.zeros((16,), jnp.float32)
        return carry

    lax.fori_loop(0, 128 * 8, zrow, 0)
    for r in range(NCH // 128):
        pltpu.sync_copy(zbuf, sh.at[pl.ds(n0 + r * 128, 128)])


HR = N_PAD // D


def _deg_body(dst_hbm, idrow_hbm, degp_hbm, dstv, hist, brow, idv, deg_sh):
    c = lax.axis_index("c")
    s = lax.axis_index("s")
    gid = c * NS + s
    row0 = gid * ROWS_PER_W
    pltpu.sync_copy(dst_hbm.at[pl.ds(row0, ROWS_PER_W)], dstv)
    pltpu.sync_copy(idrow_hbm, idv)

    def zrow(i, carry):
        r = i // 8
        k = i - r * 8
        hist[r, pl.ds(k * 16, 16)] = jnp.zeros((16,), jnp.float32)
        return carry

    lax.fori_loop(0, 128 * 8, zrow, 0)

    @pl.when(s == 0)
    def _zsh():
        pltpu.sync_copy(hist, deg_sh)

    plsc.subcore_barrier()
    one = jnp.full((16,), 1.0, jnp.float32)

    def step(t, carry):
        r = t // 8
        k = t - r * 8
        ix = dstv[r, pl.ds(k * 16, 16)]
        hi = lax.shift_right_logical(ix, 7)
        lo = lax.bitwise_and(ix, 127)
        plsc.addupdate_scatter(hist, [hi, lo], one)
        return carry

    lax.fori_loop(0, ROWS_PER_W * 8, step, 0)
    pltpu.sync_copy(hist, deg_sh.at[idv.at[0]], add=True)
    plsc.subcore_barrier()

    pltpu.sync_copy(deg_sh, hist)
    for bb in range(NCH // 128):
        base = s * NCH + bb * 128
        def brow_fill(m, carry):
            n = base + m * 16
            r = lax.shift_right_logical(n, 7)
            k = lax.bitwise_and(n, 127)
            val16 = hist[r, pl.ds(k, 16)]
            for j in range(16):
                row = jnp.broadcast_to(val16[j], (16,))
                rr = m * 16 + j
                for q in range(8):
                    brow[rr, pl.ds(q * 16, 16)] = row
            return carry

        lax.fori_loop(0, 8, brow_fill, 0)
        pltpu.sync_copy(brow, degp_hbm.at[c, pl.ds(base, 128)])


@functools.cache
def _deg_kernel():
    return pl.kernel(
        _deg_body,
        out_type=jax.ShapeDtypeStruct((NC, N_PAD, D), jnp.float32),
        mesh=plsc.VectorSubcoreMesh(core_axis_name="c", subcore_axis_name="s",
                                    num_cores=NC, num_subcores=NS),
        compiler_params=pltpu.CompilerParams(needs_layout_passes=False),
        scratch_types=[
            pltpu.VMEM((ROWS_PER_W, EB), jnp.int32),
            pltpu.VMEM((128, D), jnp.float32),
            pltpu.VMEM((128, D), jnp.float32),
            pltpu.VMEM((1, 128), jnp.int32),
            pltpu.VMEM_SHARED((128, D), jnp.float32),
        ],
    )


NBUF = 2
IB = 40


ROWS_C0 = 1280
ROWS_C1 = EROWS - ROWS_C0


def _edge_loop(row0, nblocks, src_hbm, dst_hbm, y_hbm,
               srcv, dstv, rows, gsems, ssems, agg_sh):
    for k in range(nblocks):
        pltpu.sync_copy(src_hbm.at[pl.ds(row0 + k * IB, IB)], srcv)
        pltpu.sync_copy(dst_hbm.at[pl.ds(row0 + k * IB, IB)], dstv)

        def gath(j, b):
            for h in range(0, EB, EB // 2):
                pltpu.async_copy(y_hbm.at[srcv.at[j, pl.ds(h, EB // 2)]],
                                 rows.at[b, pl.ds(h, EB // 2)], gsems[b])

        for b in range(NBUF):
            gath(b, b)

        def lap(i, carry):
            base = i * NBUF
            for b in range(NBUF):
                j = base + b
                pltpu.make_async_copy(y_hbm.at[srcv.at[j]], rows.at[b],
                                      gsems[b]).wait()
                pltpu.async_copy(rows.at[b], agg_sh.at[dstv.at[j]], ssems[b],
                                 add=True)
                nxt = j + NBUF

                @pl.when(nxt < IB)
                def _():
                    pltpu.make_async_copy(rows.at[b], agg_sh.at[dstv.at[j]],
                                          ssems[b]).wait()
                    gath(nxt, b)
            return carry

        lax.fori_loop(0, IB // NBUF, lap, 0)
        for b in range(NBUF):
            pltpu.make_async_copy(rows.at[b], agg_sh.at[dstv.at[0]],
                                  ssems[b]).wait()


def _agg_body(src_hbm, dst_hbm, y_hbm, aggp_hbm,
              srcv, dstv, rows, gsem0, gsem1, ssem0, ssem1, agg_sh):
    gsems = (gsem0, gsem1)
    ssems = (ssem0, ssem1)
    c = lax.axis_index("c")
    s = lax.axis_index("s")
    n0 = s * NCH
    _zero_slice(rows.at[0], agg_sh, n0)
    plsc.subcore_barrier()

    if ROWS_C0 > 0:
        @pl.when(c == 0)
        def _c0():
            _edge_loop(s * (ROWS_C0 // NS), ROWS_C0 // NS // IB,
                       src_hbm, dst_hbm, y_hbm, srcv, dstv, rows,
                       gsems, ssems, agg_sh)

    if ROWS_C1 > 0:
        @pl.when(c == 1)
        def _c1():
            _edge_loop(ROWS_C0 + s * (ROWS_C1 // NS), ROWS_C1 // NS // IB,
                       src_hbm, dst_hbm, y_hbm, srcv, dstv, rows,
                       gsems, ssems, agg_sh)

    plsc.subcore_barrier()
    pltpu.sync_copy(agg_sh.at[pl.ds(n0, NCH)],
                    aggp_hbm.at[c, pl.ds(n0, NCH)])


@functools.cache
def _agg_kernel():
    return pl.kernel(
        _agg_body,
        out_type=jax.ShapeDtypeStruct((NC, N_PAD, D), jnp.float32),
        mesh=plsc.VectorSubcoreMesh(core_axis_name="c", subcore_axis_name="s",
                                    num_cores=NC, num_subcores=NS),
        scratch_types=[
            pltpu.VMEM((IB, EB), jnp.int32),
            pltpu.VMEM((IB, EB), jnp.int32),
            pltpu.VMEM((NBUF, EB, D), jnp.float32),
            pltpu.SemaphoreType.DMA,
            pltpu.SemaphoreType.DMA,
            pltpu.SemaphoreType.DMA,
            pltpu.SemaphoreType.DMA,
            pltpu.VMEM_SHARED((N_PAD, D), jnp.float32),
        ],
    )


BM = 1024
BF = 1000


def _xw_body(x_ref, w_ref, xw_ref):
    xw_ref[...] = jnp.dot(x_ref[...], w_ref[...],
                          preferred_element_type=jnp.float32)


def _xw(x, W):
    return pl.pallas_call(
        _xw_body,
        grid=(N_NODES // BF,),
        in_specs=[
            pl.BlockSpec((BF, D), lambda i: (i, 0)),
            pl.BlockSpec((D, D), lambda i: (0, 0)),
        ],
        out_specs=pl.BlockSpec((BF, D), lambda i: (i, 0)),
        out_shape=jax.ShapeDtypeStruct((N_PAD, D), jnp.float32),
    )(x, W)


def _scale_body(degp_ref, xw_ref, y_ref):
    deg = degp_ref[0, :, 0:1] + degp_ref[1, :, 0:1] + 1.0
    dis = lax.rsqrt(deg)
    y_ref[...] = xw_ref[...] * dis


def _scale(degp, xw):
    return pl.pallas_call(
        _scale_body,
        grid=(N_NODES // BF,),
        in_specs=[
            pl.BlockSpec((NC, BF, D), lambda i: (0, i, 0)),
            pl.BlockSpec((BF, D), lambda i: (i, 0)),
        ],
        out_specs=pl.BlockSpec((BF, D), lambda i: (i, 0)),
        out_shape=jax.ShapeDtypeStruct((N_PAD, D), jnp.float32),
    )(degp, xw)


def _final_body(degp_ref, aggp_ref, y_ref, b_ref, out_ref):
    deg = degp_ref[0, :, 0:1] + degp_ref[1, :, 0:1] + 1.0
    dis = lax.rsqrt(deg)
    acc = aggp_ref[0] + aggp_ref[1] + y_ref[...]
    out_ref[...] = acc * dis + b_ref[...]


def _final(degp, aggp, y, b2):
    return pl.pallas_call(
        _final_body,
        grid=(N_NODES // BF,),
        in_specs=[
            pl.BlockSpec((NC, BF, D), lambda i: (0, i, 0)),
            pl.BlockSpec((NC, BF, D), lambda i: (0, i, 0)),
            pl.BlockSpec((BF, D), lambda i: (i, 0)),
            pl.BlockSpec((1, D), lambda i: (0, 0)),
        ],
        out_specs=pl.BlockSpec((BF, D), lambda i: (i, 0)),
        out_shape=jax.ShapeDtypeStruct((N_NODES, D), jnp.float32),
    )(degp, aggp, y, b2)


_PAD_IDX = np.asarray(
    N_NODES + np.arange(E_PAD - N_EDGES, dtype=np.int32) % (N_PAD - N_NODES),
    dtype=np.int32)
_ID_ROW = np.arange(128, dtype=np.int32).reshape(1, 128)


def kernel(x, edge_index, W, b):
    ei = edge_index.astype(jnp.int32)
    pad = jnp.asarray(_PAD_IDX)
    src = jnp.concatenate([ei[0], pad]).reshape(EROWS, EB)
    dst = jnp.concatenate([ei[1], pad]).reshape(EROWS, EB)

    degp = _deg_kernel()(dst, jnp.asarray(_ID_ROW))
    xw = _xw(x, W)
    y = _scale(degp, xw)
    aggp = _agg_kernel()(src, dst, y)
    out = _final(degp, aggp, y, b.reshape(1, D))
    return out
